# baseline (speedup 1.0000x reference)
import functools
import jax
import jax.numpy as jnp
from jax import lax
from jax.experimental import pallas as pl
from jax.experimental.pallas import tpu as pltpu
from jax.experimental.pallas import tpu_sc as plsc

_NW = 32
_L = 16
_N = 4 * 512 * 512
_PER_W = _N // _NW
_CH = 8192
_NCHUNK = _PER_W // _CH
_NBIN = 4096
_LOSS_HI = 16.13
_INV_W = _NBIN / _LOSS_HI
_LN2 = 0.6931471805599453
_P = (3.1969782909868627e-07, 1.4426521110421695, -0.7203866119437144,
      0.4724995251905423, -0.3231159351301159, 0.1904208313920167,
      -0.07684872596677297, 0.014778720765925785)

_mesh = plsc.VectorSubcoreMesh(core_axis_name="c", subcore_axis_name="s")


def _sc_body(pred_hbm, t_hbm, m_hbm, cnt_out, sum_out, npos_out, lpos_out,
             pred_c, t_c, m_c, cnt_v, sum_v, sc_v):
    wid = lax.axis_index("s") * 2 + lax.axis_index("c")
    base = wid * _PER_W

    zeros = jnp.zeros((_L,), jnp.float32)

    def zero_hist(i, _):
        r = i // 8
        c = (i % 8) * _L
        cnt_v[r, pl.ds(c, _L)] = zeros
        sum_v[r, pl.ds(c, _L)] = zeros
        return 0

    lax.fori_loop(0, 256, zero_hist, 0)

    ones = jnp.ones((_L,), jnp.float32)

    def chunk(ch, carry):
        off = base + ch * _CH
        pltpu.sync_copy(pred_hbm.at[pl.ds(off, _CH)], pred_c)
        pltpu.sync_copy(t_hbm.at[pl.ds(off, _CH)], t_c)
        pltpu.sync_copy(m_hbm.at[pl.ds(off, _CH)], m_c)

        def vec(i, carry2):
            npos_a, lpos_a = carry2
            p = pred_c[pl.ds(i * _L, _L)]
            t = t_c[pl.ds(i * _L, _L)]
            m = m_c[pl.ds(i * _L, _L)]
            pc = jnp.minimum(jnp.maximum(p, 1e-7), 1.0 - 1e-7)
            pos = (t * m) > 0
            neg = ((1 - t) * m) > 0
            q = jnp.where(t > 0, pc, 1.0 - pc)
            bits = plsc.bitcast(q, jnp.int32)
            e = (bits >> 23) - 127
            mant = plsc.bitcast((bits & 0x7FFFFF) | 0x3F800000, jnp.float32)
            f = mant - 1.0
            acc = jnp.full((_L,), _P[7], jnp.float32)
            for c in (_P[6], _P[5], _P[4], _P[3], _P[2], _P[1], _P[0]):
                acc = acc * f + c
            loss = -_LN2 * (e.astype(jnp.float32) + acc)
            npos_a = npos_a + jnp.where(pos, 1.0, 0.0)
            lpos_a = lpos_a + jnp.where(pos, loss, 0.0)
            b = jnp.clip((loss * _INV_W).astype(jnp.int32), 0, _NBIN - 1)
            br = b >> 7
            bc = b & 127
            plsc.addupdate_scatter(cnt_v, [br, bc], ones, mask=neg)
            plsc.addupdate_scatter(sum_v, [br, bc], loss, mask=neg)
            return npos_a, lpos_a

        return lax.fori_loop(0, _CH // _L, vec, carry)

    npos_acc, lpos_acc = lax.fori_loop(0, _NCHUNK, chunk, (zeros, zeros))
    sc_v[0, :] = npos_acc
    sc_v[1, :] = lpos_acc
    pltpu.sync_copy(cnt_v, cnt_out.at[wid])
    pltpu.sync_copy(sum_v, sum_out.at[wid])
    pltpu.sync_copy(sc_v.at[0], npos_out.at[wid])
    pltpu.sync_copy(sc_v.at[1], lpos_out.at[wid])


_sc_hist = functools.partial(
    pl.kernel, mesh=_mesh,
    out_type=(
        jax.ShapeDtypeStruct((_NW, 32, 128), jnp.float32),
        jax.ShapeDtypeStruct((_NW, 32, 128), jnp.float32),
        jax.ShapeDtypeStruct((_NW, _L), jnp.float32),
        jax.ShapeDtypeStruct((_NW, _L), jnp.float32),
    ),
    scratch_types=[
        pltpu.VMEM((_CH,), jnp.float32),
        pltpu.VMEM((_CH,), jnp.int32),
        pltpu.VMEM((_CH,), jnp.int32),
        pltpu.VMEM((32, 128), jnp.float32),
        pltpu.VMEM((32, 128), jnp.float32),
        pltpu.VMEM((2, _L), jnp.float32),
    ],
    compiler_params=pltpu.CompilerParams(needs_layout_passes=False),
)(_sc_body)


def _tc_select_body(cnt_ref, sum_ref, npw_ref, lpw_ref, out_ref):
    C2 = jnp.sum(cnt_ref[...], axis=0)
    S2 = jnp.sum(sum_ref[...], axis=0)
    npos = jnp.sum(npw_ref[...])
    lpos = jnp.sum(lpw_ref[...])
    hp = jax.lax.Precision.HIGHEST
    M1 = (lax.broadcasted_iota(jnp.int32, (128, 128), 0)
          >= lax.broadcasted_iota(jnp.int32, (128, 128), 1)).astype(jnp.float32)
    SufC = jnp.dot(C2, M1, preferred_element_type=jnp.float32, precision=hp)
    SufS = jnp.dot(S2, M1, preferred_element_type=jnp.float32, precision=hp)
    Arr = (lax.broadcasted_iota(jnp.int32, (32, 32), 1)
           > lax.broadcasted_iota(jnp.int32, (32, 32), 0)).astype(jnp.float32)
    RowC = jnp.dot(Arr, SufC[:, 0:1], preferred_element_type=jnp.float32,
                   precision=hp)
    RowS = jnp.dot(Arr, SufS[:, 0:1], preferred_element_type=jnp.float32,
                   precision=hp)
    C_geq = RowC + SufC
    C_above = C_geq - C2
    S_above = RowS + SufS - S2
    nneg = jnp.sum(C2)
    k = jnp.where(npos > 0.0, jnp.minimum(nneg, 3.0 * npos), 100.0)
    k_eff = jnp.minimum(k, nneg)
    sel = jnp.logical_and(C_above < k_eff, C_geq >= k_eff)
    self32 = jnp.where(sel, 1.0, 0.0) * jnp.where(k_eff > 0.0, 1.0, 0.0)
    cnt_sel = jnp.sum(self32 * C2)
    sum_sel = jnp.sum(self32 * S2)
    C_a = jnp.sum(self32 * C_above)
    S_a = jnp.sum(self32 * S_above)
    mean_sel = sum_sel / jnp.maximum(cnt_sel, 1.0)
    loss_neg = jnp.where(k_eff > 0.0, S_a + (k_eff - C_a) * mean_sel, 0.0)
    loss_neg = loss_neg + jnp.where(k > nneg, (k - nneg) * -1e30, 0.0)
    out_ref[0, 0] = (lpos + loss_neg) / (npos + k)


def _tc_select(cnt3, sum3, npw, lpw):
    out = pl.pallas_call(
        _tc_select_body,
        out_specs=pl.BlockSpec(memory_space=pltpu.SMEM),
        out_shape=jax.ShapeDtypeStruct((1, 1), jnp.float32),
    )(cnt3, sum3, npw, lpw)
    return out[0, 0]


def kernel(pred, target, train_mask):
    cnt3, sum3, npw, lpw = _sc_hist(
        pred.reshape(-1), target.reshape(-1), train_mask.reshape(-1))
    return _tc_select(cnt3, sum3, npw, lpw)


# speedup vs baseline: 1.0764x; 1.0764x over previous
import functools
import jax, jax.numpy as jnp
from jax import lax
from jax.experimental import pallas as pl
from jax.experimental.pallas import tpu as pltpu
from jax.experimental.pallas import tpu_sc as plsc

NW = 32
L = 16
N = 4 * 512 * 512
PER_W = N // NW
CH = 8192
NCHUNK = PER_W // CH
NBIN = 4096
NROW = 40
POSROW = 32
LOSS_HI = 16.13
INV_W = NBIN / LOSS_HI
NEG_LN2 = -0.6931471805599453
P5 = (1.4390929995222734e-05, 1.441592077206554, -0.7072534335743862,
      0.4115614823105297, -0.18983244652673942, 0.04392862784796933)

_mesh = plsc.VectorSubcoreMesh(core_axis_name="c", subcore_axis_name="s")


def _sc_body(pred_hbm, t_hbm, m_hbm, cnt_out, sum_out, pred_c, t_c, m_c,
             cnt_v, sum_v):
    wid = lax.axis_index("s") * 2 + lax.axis_index("c")
    base = wid * PER_W

    zeros = jnp.zeros((L,), jnp.float32)

    def zero_hist(i, _):
        r = i // 8
        c = (i % 8) * L
        cnt_v[r, pl.ds(c, L)] = zeros
        sum_v[r, pl.ds(c, L)] = zeros
        return 0

    lax.fori_loop(0, NROW * 8, zero_hist, 0, unroll=4)

    ones = jnp.ones((L,), jnp.float32)
    lanes = lax.iota(jnp.int32, L)

    def chunk(ch, _):
        off = base + ch * CH
        pltpu.sync_copy(pred_hbm.at[pl.ds(off, CH)], pred_c)
        pltpu.sync_copy(t_hbm.at[pl.ds(off, CH)], t_c)
        pltpu.sync_copy(m_hbm.at[pl.ds(off, CH)], m_c)

        def vec(i, _):
            p = pred_c[pl.ds(i * L, L)]
            t = t_c[pl.ds(i * L, L)]
            m = m_c[pl.ds(i * L, L)]
            tpos = t > 0
            q = jnp.maximum(jnp.where(tpos, p, 1.0 - p), 1e-7)
            bits = plsc.bitcast(q, jnp.int32)
            e = (bits >> 23) - 127
            mant = plsc.bitcast((bits & 0x7FFFFF) | 0x3F800000, jnp.float32)
            f = mant - 1.0
            acc = jnp.full((L,), P5[5], jnp.float32)
            for c in (P5[4], P5[3], P5[2], P5[1], P5[0]):
                acc = acc * f + c
            loss = NEG_LN2 * (e.astype(jnp.float32) + acc)
            b = jnp.clip((loss * INV_W).astype(jnp.int32), 0, NBIN - 1)
            br = jnp.where(tpos, POSROW, b >> 7)
            bc = jnp.where(tpos, lanes, b & 127)
            msk = m > 0
            plsc.addupdate_scatter(cnt_v, [br, bc], ones, mask=msk)
            plsc.addupdate_scatter(sum_v, [br, bc], loss, mask=msk)
            return 0

        lax.fori_loop(0, CH // L, vec, 0, unroll=4)
        return 0

    lax.fori_loop(0, NCHUNK, chunk, 0)
    pltpu.sync_copy(cnt_v, cnt_out.at[wid])
    pltpu.sync_copy(sum_v, sum_out.at[wid])


_sc_hist = functools.partial(
    pl.kernel, mesh=_mesh,
    out_type=(
        jax.ShapeDtypeStruct((NW, NROW, 128), jnp.float32),
        jax.ShapeDtypeStruct((NW, NROW, 128), jnp.float32),
    ),
    scratch_types=[
        pltpu.VMEM((CH,), jnp.float32),
        pltpu.VMEM((CH,), jnp.int32),
        pltpu.VMEM((CH,), jnp.int32),
        pltpu.VMEM((NROW, 128), jnp.float32),
        pltpu.VMEM((NROW, 128), jnp.float32),
    ],
    compiler_params=pltpu.CompilerParams(needs_layout_passes=False),
)(_sc_body)


def _tc_select_body(cnt_ref, sum_ref, out_ref):
    A = jnp.sum(cnt_ref[...], axis=0)
    B = jnp.sum(sum_ref[...], axis=0)
    C2 = A[0:32, :]
    S2 = B[0:32, :]
    npos = jnp.sum(A[32:40, :])
    lpos = jnp.sum(B[32:40, :])
    hp = jax.lax.Precision.HIGHEST
    M1 = (lax.broadcasted_iota(jnp.int32, (128, 128), 0)
          >= lax.broadcasted_iota(jnp.int32, (128, 128), 1)).astype(jnp.float32)
    SufC = jnp.dot(C2, M1, preferred_element_type=jnp.float32, precision=hp)
    SufS = jnp.dot(S2, M1, preferred_element_type=jnp.float32, precision=hp)
    Arr = (lax.broadcasted_iota(jnp.int32, (32, 32), 1)
           > lax.broadcasted_iota(jnp.int32, (32, 32), 0)).astype(jnp.float32)
    RowC = jnp.dot(Arr, SufC[:, 0:1], preferred_element_type=jnp.float32,
                   precision=hp)
    RowS = jnp.dot(Arr, SufS[:, 0:1], preferred_element_type=jnp.float32,
                   precision=hp)
    C_geq = RowC + SufC
    C_above = C_geq - C2
    S_above = RowS + SufS - S2
    nneg = jnp.sum(C2)
    k = jnp.where(npos > 0.0, jnp.minimum(nneg, 3.0 * npos), 100.0)
    k_eff = jnp.minimum(k, nneg)
    sel = jnp.logical_and(C_above < k_eff, C_geq >= k_eff)
    self32 = jnp.where(sel, 1.0, 0.0) * jnp.where(k_eff > 0.0, 1.0, 0.0)
    cnt_sel = jnp.sum(self32 * C2)
    sum_sel = jnp.sum(self32 * S2)
    C_a = jnp.sum(self32 * C_above)
    S_a = jnp.sum(self32 * S_above)
    mean_sel = sum_sel / jnp.maximum(cnt_sel, 1.0)
    loss_neg = jnp.where(k_eff > 0.0, S_a + (k_eff - C_a) * mean_sel, 0.0)
    loss_neg = loss_neg + jnp.where(k > nneg, (k - nneg) * -1e30, 0.0)
    out_ref[0, 0] = (lpos + loss_neg) / (npos + k)


def _tc_select(cnt3, sum3):
    out = pl.pallas_call(
        _tc_select_body,
        out_specs=pl.BlockSpec(memory_space=pltpu.SMEM),
        out_shape=jax.ShapeDtypeStruct((1, 1), jnp.float32),
    )(cnt3, sum3)
    return out[0, 0]


def kernel_sc(pred, target, train_mask):
    cnt3, sum3 = _sc_hist(
        pred.reshape(-1), target.reshape(-1), train_mask.reshape(-1))
    return _tc_select(cnt3, sum3)


def kernel(pred, target, train_mask):
    return kernel_sc(pred, target, train_mask)


# speedup vs baseline: 1.6535x; 1.5361x over previous
import functools
import jax
import jax.numpy as jnp
from jax import lax
from jax.experimental import pallas as pl
from jax.experimental.pallas import tpu as pltpu
from jax.experimental.pallas import tpu_sc as plsc

NW = 32
L = 16
N = 4 * 512 * 512
PER_W = N // NW
CH = 8192
NCHUNK = PER_W // CH
KEY_BASE = 103 * 64
NROW = 32
POS_OFF = 16

_mesh = plsc.VectorSubcoreMesh(core_axis_name="c", subcore_axis_name="s")


def _sc_body(pred_hbm, t_hbm, m_hbm, cnt_out, sum_out, pred_c, t_c, m_c,
             cnt_v, sum_v):
    wid = lax.axis_index("s") * 2 + lax.axis_index("c")
    base = wid * PER_W

    zeros = jnp.zeros((L,), jnp.float32)

    def zero_hist(i, _):
        r = i // 8
        c = (i % 8) * L
        cnt_v[r, pl.ds(c, L)] = zeros
        sum_v[r, pl.ds(c, L)] = zeros
        return 0

    lax.fori_loop(0, NROW * 8, zero_hist, 0, unroll=4)

    ones = jnp.ones((L,), jnp.float32)

    def chunk(ch, _):
        off = base + ch * CH
        pltpu.sync_copy(pred_hbm.at[pl.ds(off, CH)], pred_c)
        pltpu.sync_copy(t_hbm.at[pl.ds(off, CH)], t_c)
        pltpu.sync_copy(m_hbm.at[pl.ds(off, CH)], m_c)

        def vec(i, _):
            p = pred_c[pl.ds(i * L, L)]
            t = t_c[pl.ds(i * L, L)]
            m = m_c[pl.ds(i * L, L)]
            tpos = t > 0
            q = jnp.minimum(jnp.maximum(jnp.where(tpos, p, 1.0 - p), 1e-7),
                            1.0 - 1e-7)
            key = (plsc.bitcast(q, jnp.int32) >> 17) - KEY_BASE
            br = (key >> 7) + jnp.where(tpos, POS_OFF, 0)
            bc = key & 127
            msk = m > 0
            plsc.addupdate_scatter(cnt_v, [br, bc], ones, mask=msk)
            plsc.addupdate_scatter(sum_v, [br, bc], q, mask=msk)
            return 0

        lax.fori_loop(0, CH // L, vec, 0, unroll=4)
        return 0

    lax.fori_loop(0, NCHUNK, chunk, 0)
    pltpu.sync_copy(cnt_v, cnt_out.at[wid])
    pltpu.sync_copy(sum_v, sum_out.at[wid])


_sc_hist = functools.partial(
    pl.kernel, mesh=_mesh,
    out_type=(
        jax.ShapeDtypeStruct((NW, NROW, 128), jnp.float32),
        jax.ShapeDtypeStruct((NW, NROW, 128), jnp.float32),
    ),
    scratch_types=[
        pltpu.VMEM((CH,), jnp.float32),
        pltpu.VMEM((CH,), jnp.int32),
        pltpu.VMEM((CH,), jnp.int32),
        pltpu.VMEM((NROW, 128), jnp.float32),
        pltpu.VMEM((NROW, 128), jnp.float32),
    ],
    compiler_params=pltpu.CompilerParams(needs_layout_passes=False),
)(_sc_body)


def _tc_select_body(cnt_ref, sum_ref, out_ref):
    A = jnp.sum(cnt_ref[...], axis=0)
    Q = jnp.sum(sum_ref[...], axis=0)
    C2 = A[0:16, :]
    Q2 = Q[0:16, :]
    posC = A[16:32, :]
    posQ = Q[16:32, :]
    npos = jnp.sum(posC)
    qm_pos = jnp.where(posC > 0.0, posQ / jnp.maximum(posC, 1.0), 0.5)
    lpos = jnp.sum(posC * -jnp.log(qm_pos))
    qm_neg = jnp.where(C2 > 0.0, Q2 / jnp.maximum(C2, 1.0), 0.5)
    S2 = C2 * -jnp.log(qm_neg)
    hp = jax.lax.Precision.HIGHEST
    M1 = (lax.broadcasted_iota(jnp.int32, (128, 128), 0)
          <= lax.broadcasted_iota(jnp.int32, (128, 128), 1)).astype(jnp.float32)
    PrefC = jnp.dot(C2, M1, preferred_element_type=jnp.float32, precision=hp)
    PrefS = jnp.dot(S2, M1, preferred_element_type=jnp.float32, precision=hp)
    Arr = (lax.broadcasted_iota(jnp.int32, (16, 16), 1)
           < lax.broadcasted_iota(jnp.int32, (16, 16), 0)).astype(jnp.float32)
    RowC = jnp.dot(Arr, PrefC[:, 127:128], preferred_element_type=jnp.float32,
                   precision=hp)
    RowS = jnp.dot(Arr, PrefS[:, 127:128], preferred_element_type=jnp.float32,
                   precision=hp)
    C_geq = RowC + PrefC
    C_above = C_geq - C2
    S_above = RowS + PrefS - S2
    nneg = jnp.sum(C2)
    k = jnp.where(npos > 0.0, jnp.minimum(nneg, 3.0 * npos), 100.0)
    k_eff = jnp.minimum(k, nneg)
    sel = jnp.logical_and(C_above < k_eff, C_geq >= k_eff)
    self32 = jnp.where(sel, 1.0, 0.0) * jnp.where(k_eff > 0.0, 1.0, 0.0)
    cnt_sel = jnp.sum(self32 * C2)
    sum_sel = jnp.sum(self32 * S2)
    C_a = jnp.sum(self32 * C_above)
    S_a = jnp.sum(self32 * S_above)
    mean_sel = sum_sel / jnp.maximum(cnt_sel, 1.0)
    loss_neg = jnp.where(k_eff > 0.0, S_a + (k_eff - C_a) * mean_sel, 0.0)
    loss_neg = loss_neg + jnp.where(k > nneg, (k - nneg) * -1e30, 0.0)
    out_ref[0, 0] = (lpos + loss_neg) / (npos + k)


def _tc_select(cnt3, sum3):
    out = pl.pallas_call(
        _tc_select_body,
        out_specs=pl.BlockSpec(memory_space=pltpu.SMEM),
        out_shape=jax.ShapeDtypeStruct((1, 1), jnp.float32),
    )(cnt3, sum3)
    return out[0, 0]


def kernel(pred, target, train_mask):
    cnt3, sum3 = _sc_hist(
        pred.reshape(-1), target.reshape(-1), train_mask.reshape(-1))
    return _tc_select(cnt3, sum3)
